# baseline (speedup 1.0000x reference)
import functools

import jax
import jax.numpy as jnp
import numpy as np
from jax.experimental import pallas as pl
from jax.experimental.pallas import tpu as pltpu

_DIM = 96
_H = 128
_EMB_NAMES = ("ds1", "ds2", "m0", "m1", "m2", "b0", "b1", "b2")


def _shift(a, dy, dx):
    if dy == 1:
        a = jnp.concatenate([a[..., 1:, :], jnp.zeros_like(a[..., :1, :])], axis=-2)
    elif dy == -1:
        a = jnp.concatenate([jnp.zeros_like(a[..., :1, :]), a[..., :-1, :]], axis=-2)
    if dx == 1:
        a = jnp.concatenate([a[..., :, 1:], jnp.zeros_like(a[..., :, :1])], axis=-1)
    elif dx == -1:
        a = jnp.concatenate([jnp.zeros_like(a[..., :, :1]), a[..., :, :-1]], axis=-1)
    return a


def _conv3x3_scalar(a, wvec):
    rows = {-1: _shift(a, -1, 0), 0: a, 1: _shift(a, 1, 0)}
    out = None
    for dx in (-1, 0, 1):
        t_dx = None
        for dy in (-1, 0, 1):
            term = wvec[(dy + 1) * 3 + (dx + 1)] * rows[dy]
            t_dx = term if t_dx is None else t_dx + term
        t_dx = _shift(t_dx, 0, dx)
        out = t_dx if out is None else out + t_dx
    return out


def _dwconv3x3(x, w, bias, rows=None):
    if rows is None:
        rows = {-1: _shift(x, -1, 0), 0: x, 1: _shift(x, 1, 0)}
    out = bias[:, :, None]
    for dx in (-1, 0, 1):
        t_dx = None
        for dy in (-1, 0, 1):
            term = w[:, (dy + 1) * 3 + (dx + 1)][:, None, None] * rows[dy]
            t_dx = term if t_dx is None else t_dx + term
        out = out + _shift(t_dx, 0, dx)
    return out


def _leaky(a, slope):
    return jnp.where(a > 0, a, slope * a)


def _mofe_kernel(x_ref, c1w_ref, c1b_ref, c2w_ref, c2b_ref, embw_ref, embb_ref,
                 fusew_ref, fc0w_ref, fc0b_ref, fc1w_ref, fc1b_ref,
                 s_ref, st_ref, u_ref, ut_ref, fw_ref, fb_ref,
                 out_ref, coef_ref, acc_ref, flag_ref):
    b = pl.program_id(0)
    g = pl.program_id(1)
    e = pl.program_id(2)
    x = x_ref[0]

    @pl.when(jnp.logical_and(g == 0, e == 0))
    def _gating():
        pooled = jnp.max(x, axis=(1, 2)) + jnp.mean(x, axis=(1, 2))
        for gg in range(3):
            hs = []
            ns = []
            for j in range(4):
                hj = jnp.sum(fc1w_ref[gg, j, :] * pooled) + fc1b_ref[gg, j]
                zj = jnp.sum(fc0w_ref[gg, j, :] * pooled) + fc0b_ref[gg, j]
                nj = jnp.maximum(zj, 0.0) + jnp.log(1.0 + jnp.exp(-jnp.abs(zj)))
                hs.append(jnp.where(hj > 0, hj, 0.2 * hj))
                ns.append(nj)
            nm = (ns[0] + ns[1] + ns[2] + ns[3]) * 0.25
            var = ((ns[0] - nm) ** 2 + (ns[1] - nm) ** 2 +
                   (ns[2] - nm) ** 2 + (ns[3] - nm) ** 2) / 3.0
            std = jnp.sqrt(var) + 1e-6
            s = [hs[j] + (ns[j] - nm) / std for j in range(4)]
            keeps = []
            for j in range(4):
                rank = jnp.float32(0.0)
                for k in range(4):
                    if k == j:
                        continue
                    gt = (s[k] > s[j]).astype(jnp.float32)
                    eq = jnp.logical_and(s[k] == s[j], k < j).astype(jnp.float32)
                    rank = rank + gt + eq
                keeps.append(rank <= 2.0)
            neg = jnp.float32(-1e30)
            m01 = jnp.maximum(jnp.where(keeps[0], hs[0], neg),
                              jnp.where(keeps[1], hs[1], neg))
            m23 = jnp.maximum(jnp.where(keeps[2], hs[2], neg),
                              jnp.where(keeps[3], hs[3], neg))
            m = jnp.maximum(m01, m23)
            zs = [jnp.where(keeps[j], jnp.exp(hs[j] - m), 0.0) for j in range(4)]
            ztot = zs[0] + zs[1] + zs[2] + zs[3]
            for j in range(4):
                coef_ref[gg, j] = zs[j] / ztot

    coeff = coef_ref[g, e]

    @pl.when(jnp.logical_and(g == 0, e == 0))
    def _init_out():
        out_ref[0] = jnp.broadcast_to(fb_ref[...][:, :, None], (_DIM, _H, _H))

    @pl.when(e == 0)
    def _init_acc():
        flag_ref[0] = 0

    @pl.when(coeff != 0.0)
    def _expert():
        _expert_body(x, g, coeff, c1w_ref, c1b_ref, c2w_ref, c2b_ref,
                     embw_ref, embb_ref, fusew_ref, s_ref, st_ref, u_ref,
                     ut_ref, acc_ref, flag_ref)

    @pl.when(e == 3)
    def _fuse():
        for k in range(4):
            rows = slice(k * (_H // 4), (k + 1) * (_H // 4))
            a = acc_ref[:, rows, :].reshape(_DIM, (_H // 4) * _H)
            r = jnp.dot(fw_ref[0], a, preferred_element_type=jnp.float32)
            out_ref[0, :, rows, :] = (out_ref[0, :, rows, :] +
                                      r.reshape(_DIM, _H // 4, _H))


def _expert_body(x, g, coeff, c1w_ref, c1b_ref, c2w_ref, c2b_ref, embw_ref,
                 embb_ref, fusew_ref, s_ref, st_ref, u_ref, ut_ref, acc_ref,
                 flag_ref):
    e = pl.program_id(2)
    h = _dwconv3x3(x, c1w_ref[0], c1b_ref[0])

    avg = jnp.mean(h, axis=0)
    mx = jnp.max(h, axis=0)
    msq = jnp.mean(h * h, axis=0)
    std = jnp.sqrt(jnp.maximum(msq - avg * avg, 1e-6))
    f1 = jnp.where(g == 0, avg, std)
    f2 = jnp.where(g == 2, avg, mx)

    row = g * 4 + e

    def emb_w(name, t):
        return embw_ref[row, _EMB_NAMES.index(name) * 9 + t]

    def emb_b(name):
        return embb_ref[row, _EMB_NAMES.index(name)]

    def conv_named(a, name):
        return _conv3x3_scalar(a, [emb_w(name, t) for t in range(9)])

    S = s_ref[...]
    St = st_ref[...]
    y1 = conv_named(f1, "ds1")
    y2 = conv_named(f2, "ds2")
    f1d = jnp.dot(S, jnp.dot(y1, St, preferred_element_type=jnp.float32),
                  preferred_element_type=jnp.float32) + emb_b("ds1")
    f2d = jnp.dot(S, jnp.dot(y2, St, preferred_element_type=jnp.float32),
                  preferred_element_type=jnp.float32) + emb_b("ds2")

    def m_conv(z):
        z = _leaky(conv_named(z, "m0") + emb_b("m0"), 0.01)
        z = _leaky(conv_named(z, "m1") + emb_b("m1"), 0.01)
        return jnp.tanh(conv_named(z, "m2") + emb_b("m2"))

    def b_conv(z):
        z = _leaky(conv_named(z, "b0") + emb_b("b0"), 0.01)
        z = _leaky(conv_named(z, "b1") + emb_b("b1"), 0.01)
        return conv_named(z, "b2") + emb_b("b2")

    w1 = m_conv(f1d)
    b1 = b_conv(f1d)
    w2 = m_conv(f2d)
    b2 = b_conv(f2d)
    mod2 = f2d * w1 + b1
    mod1 = f1d * w2 + b2
    fused = (_conv3x3_scalar(mod1, [fusew_ref[row, t] for t in range(9)]) +
             _conv3x3_scalar(mod2, [fusew_ref[row, 9 + t] for t in range(9)]) +
             embb_ref[row, 8])

    U = u_ref[...]
    Ut = ut_ref[...]
    up = jnp.dot(U, jnp.dot(fused, Ut, preferred_element_type=jnp.float32),
                 preferred_element_type=jnp.float32)
    mask = 1.0 / (1.0 + jnp.exp(-up))
    e_in = jnp.maximum(h * mask[None, :, :], 0.0)

    eo = _dwconv3x3(e_in, coeff * c2w_ref[0], coeff * c2b_ref[0])
    first = flag_ref[0]

    @pl.when(first == 0)
    def _set():
        acc_ref[...] = eo

    @pl.when(first != 0)
    def _add():
        acc_ref[...] = acc_ref[...] + eo

    flag_ref[0] = 1


def _fusion_kernel(a_ref, w_ref, b_ref, o_ref):
    a = a_ref[0]
    o_ref[0] = jnp.dot(w_ref[...], a,
                       preferred_element_type=jnp.float32) + b_ref[...]


def _make_resize_mats():
    r = np.arange(_H)[:, None]
    c = np.arange(_H // 2)[None, :]
    k = r // 2
    even = (r % 2 == 0)
    U = (np.where(even & (c == k - 1), 0.25, 0.0) +
         np.where(even & (c == k), 0.75, 0.0) +
         np.where(~even & (c == k), 0.75, 0.0) +
         np.where(~even & (c == k + 1), 0.25, 0.0)).astype(np.float32)
    U[0, 0] += 0.25
    U[_H - 1, _H // 2 - 1] += 0.25
    rs = np.arange(_H // 2)[:, None]
    cs = np.arange(_H)[None, :]
    S = (cs == 2 * rs).astype(np.float32)
    return U, S


@jax.jit
def kernel(x, params):
    B = x.shape[0]
    groups = params["groups"]
    gates = params["gates"]

    c1w = jnp.stack([groups[g][e]["c1_w"].reshape(_DIM, 9)
                     for g in range(3) for e in range(4)])
    c1b = jnp.stack([groups[g][e]["c1_b"]
                     for g in range(3) for e in range(4)])[:, :, None]
    c2w = jnp.stack([groups[g][e]["c2_w"].reshape(_DIM, 9)
                     for g in range(3) for e in range(4)])
    c2b = jnp.stack([groups[g][e]["c2_b"]
                     for g in range(3) for e in range(4)])[:, :, None]
    embw = jnp.stack([
        jnp.concatenate([groups[g][e]["emb"][nm + "_w"].reshape(9)
                         for nm in _EMB_NAMES])
        for g in range(3) for e in range(4)])
    embb = jnp.stack([
        jnp.concatenate([jnp.stack([groups[g][e]["emb"][nm + "_b"][0]
                                    for nm in _EMB_NAMES]),
                         groups[g][e]["emb"]["fuse_b"]])
        for g in range(3) for e in range(4)])
    fusew = jnp.stack([groups[g][e]["emb"]["fuse_w"].reshape(18)
                       for g in range(3) for e in range(4)])
    fc0w = jnp.stack([gates[g]["fc0_w"] for g in range(3)])
    fc0b = jnp.stack([gates[g]["fc0_b"] for g in range(3)])
    fc1w = jnp.stack([gates[g]["fc1_w"] for g in range(3)])
    fc1b = jnp.stack([gates[g]["fc1_b"] for g in range(3)])

    U, S = _make_resize_mats()
    U = jnp.asarray(U)
    S = jnp.asarray(S)

    vm = pltpu.VMEM
    sm = pltpu.SMEM
    zero5 = lambda b, g, e: (0, 0, 0, 0, 0)

    inter = pl.pallas_call(
        _mofe_kernel,
        grid=(B, 3, 4),
        in_specs=[
            pl.BlockSpec((1, _DIM, _H, _H), lambda b, g, e: (b, 0, 0, 0)),
            pl.BlockSpec((1, _DIM, 9), lambda b, g, e: (g * 4 + e, 0, 0)),
            pl.BlockSpec((1, _DIM, 1), lambda b, g, e: (g * 4 + e, 0, 0)),
            pl.BlockSpec((1, _DIM, 9), lambda b, g, e: (g * 4 + e, 0, 0)),
            pl.BlockSpec((1, _DIM, 1), lambda b, g, e: (g * 4 + e, 0, 0)),
            pl.BlockSpec(memory_space=sm),
            pl.BlockSpec(memory_space=sm),
            pl.BlockSpec(memory_space=sm),
            pl.BlockSpec(memory_space=vm),
            pl.BlockSpec(memory_space=sm),
            pl.BlockSpec(memory_space=vm),
            pl.BlockSpec(memory_space=sm),
            pl.BlockSpec(memory_space=vm),
            pl.BlockSpec(memory_space=vm),
            pl.BlockSpec(memory_space=vm),
            pl.BlockSpec(memory_space=vm),
            pl.BlockSpec((1, _DIM, _DIM), lambda b, g, e: (g, 0, 0)),
            pl.BlockSpec(memory_space=vm),
        ],
        out_specs=pl.BlockSpec((1, _DIM, _H, _H),
                               lambda b, g, e: (b, 0, 0, 0)),
        out_shape=jax.ShapeDtypeStruct((B, _DIM, _H, _H), jnp.float32),
        scratch_shapes=[pltpu.SMEM((3, 4), jnp.float32),
                        pltpu.VMEM((_DIM, _H, _H), jnp.float32),
                        pltpu.SMEM((1,), jnp.int32)],
        compiler_params=pltpu.CompilerParams(
            dimension_semantics=("parallel", "arbitrary", "arbitrary"),
            vmem_limit_bytes=100 * 1024 * 1024),
    )(x, c1w, c1b, c2w, c2b, embw, embb, fusew,
      fc0w, fc0b, fc1w, fc1b, S, S.T, U, U.T,
      params["fusion_w"].reshape(_DIM, 3, _DIM).transpose(1, 0, 2),
      params["fusion_b"].reshape(_DIM, 1))

    return inter


# speedup vs baseline: 1.0154x; 1.0154x over previous
import functools

import jax
import jax.numpy as jnp
import numpy as np
from jax.experimental import pallas as pl
from jax.experimental.pallas import tpu as pltpu

_DIM = 96
_H = 128
_EMB_NAMES = ("ds1", "ds2", "m0", "m1", "m2", "b0", "b1", "b2")


def _shift(a, dy, dx):
    if dy == 1:
        a = jnp.concatenate([a[..., 1:, :], jnp.zeros_like(a[..., :1, :])], axis=-2)
    elif dy == -1:
        a = jnp.concatenate([jnp.zeros_like(a[..., :1, :]), a[..., :-1, :]], axis=-2)
    if dx == 1:
        a = jnp.concatenate([a[..., :, 1:], jnp.zeros_like(a[..., :, :1])], axis=-1)
    elif dx == -1:
        a = jnp.concatenate([jnp.zeros_like(a[..., :, :1]), a[..., :, :-1]], axis=-1)
    return a


def _conv3x3_scalar(a, wvec):
    rows = {-1: _shift(a, -1, 0), 0: a, 1: _shift(a, 1, 0)}
    out = None
    for dx in (-1, 0, 1):
        t_dx = None
        for dy in (-1, 0, 1):
            term = wvec[(dy + 1) * 3 + (dx + 1)] * rows[dy]
            t_dx = term if t_dx is None else t_dx + term
        t_dx = _shift(t_dx, 0, dx)
        out = t_dx if out is None else out + t_dx
    return out


def _dwconv3x3(x, w, bias, rows=None):
    if rows is None:
        rows = {-1: _shift(x, -1, 0), 0: x, 1: _shift(x, 1, 0)}
    out = bias[:, :, None]
    for dx in (-1, 0, 1):
        t_dx = None
        for dy in (-1, 0, 1):
            term = w[:, (dy + 1) * 3 + (dx + 1)][:, None, None] * rows[dy]
            t_dx = term if t_dx is None else t_dx + term
        out = out + _shift(t_dx, 0, dx)
    return out


def _leaky(a, slope):
    return jnp.where(a > 0, a, slope * a)


def _mofe_kernel(x_ref, c1w_ref, c1b_ref, c2w_ref, c2b_ref, embw_ref, embb_ref,
                 fusew_ref, fc0w_ref, fc0b_ref, fc1w_ref, fc1b_ref,
                 s_ref, st_ref, u_ref, ut_ref, fw_ref, fb_ref,
                 out_ref, coef_ref, acc_ref, flag_ref):
    b = pl.program_id(0)
    g = pl.program_id(1)
    e = pl.program_id(2)
    x = x_ref[0]

    @pl.when(jnp.logical_and(g == 0, e == 0))
    def _gating():
        pooled = jnp.max(x, axis=(1, 2)) + jnp.mean(x, axis=(1, 2))
        for gg in range(3):
            hs = []
            ns = []
            for j in range(4):
                hj = jnp.sum(fc1w_ref[gg, j, :] * pooled) + fc1b_ref[gg, j]
                zj = jnp.sum(fc0w_ref[gg, j, :] * pooled) + fc0b_ref[gg, j]
                nj = jnp.maximum(zj, 0.0) + jnp.log(1.0 + jnp.exp(-jnp.abs(zj)))
                hs.append(jnp.where(hj > 0, hj, 0.2 * hj))
                ns.append(nj)
            nm = (ns[0] + ns[1] + ns[2] + ns[3]) * 0.25
            var = ((ns[0] - nm) ** 2 + (ns[1] - nm) ** 2 +
                   (ns[2] - nm) ** 2 + (ns[3] - nm) ** 2) / 3.0
            std = jnp.sqrt(var) + 1e-6
            s = [hs[j] + (ns[j] - nm) / std for j in range(4)]
            keeps = []
            for j in range(4):
                rank = jnp.float32(0.0)
                for k in range(4):
                    if k == j:
                        continue
                    gt = (s[k] > s[j]).astype(jnp.float32)
                    eq = jnp.logical_and(s[k] == s[j], k < j).astype(jnp.float32)
                    rank = rank + gt + eq
                keeps.append(rank <= 2.0)
            neg = jnp.float32(-1e30)
            m01 = jnp.maximum(jnp.where(keeps[0], hs[0], neg),
                              jnp.where(keeps[1], hs[1], neg))
            m23 = jnp.maximum(jnp.where(keeps[2], hs[2], neg),
                              jnp.where(keeps[3], hs[3], neg))
            m = jnp.maximum(m01, m23)
            zs = [jnp.where(keeps[j], jnp.exp(hs[j] - m), 0.0) for j in range(4)]
            ztot = zs[0] + zs[1] + zs[2] + zs[3]
            for j in range(4):
                coef_ref[gg, j] = zs[j] / ztot

    coeff = coef_ref[g, e]

    @pl.when(jnp.logical_and(g == 0, e == 0))
    def _init_out():
        out_ref[0] = jnp.broadcast_to(fb_ref[...][:, :, None], (_DIM, _H, _H))

    @pl.when(e == 0)
    def _init_acc():
        flag_ref[0] = 0

    @pl.when(coeff != 0.0)
    def _expert():
        _expert_body(x, g, coeff, c1w_ref, c1b_ref, c2w_ref, c2b_ref,
                     embw_ref, embb_ref, fusew_ref, s_ref, st_ref, u_ref,
                     ut_ref, acc_ref, flag_ref)

    @pl.when(e == 3)
    def _fuse():
        for k in range(4):
            rows = slice(k * (_H // 4), (k + 1) * (_H // 4))
            a = acc_ref[:, rows, :].reshape(_DIM, (_H // 4) * _H)
            r = jnp.dot(fw_ref[0], a, preferred_element_type=jnp.float32)
            out_ref[0, :, rows, :] = (out_ref[0, :, rows, :] +
                                      r.reshape(_DIM, _H // 4, _H))


def _expert_body(x, g, coeff, c1w_ref, c1b_ref, c2w_ref, c2b_ref, embw_ref,
                 embb_ref, fusew_ref, s_ref, st_ref, u_ref, ut_ref, acc_ref,
                 flag_ref):
    e = pl.program_id(2)
    h = _dwconv3x3(x, c1w_ref[0], c1b_ref[0])

    avg = jnp.mean(h, axis=0)
    mx = jnp.max(h, axis=0)
    msq = jnp.mean(h * h, axis=0)
    std = jnp.sqrt(jnp.maximum(msq - avg * avg, 1e-6))
    f1 = jnp.where(g == 0, avg, std)
    f2 = jnp.where(g == 2, avg, mx)

    row = g * 4 + e

    def emb_w(name, t):
        return embw_ref[row, _EMB_NAMES.index(name) * 9 + t]

    def emb_b(name):
        return embb_ref[row, _EMB_NAMES.index(name)]

    def wstack(names, t):
        return jnp.stack([emb_w(nm, t) for nm in names])[:, None, None]

    def bstack(names):
        return jnp.stack([emb_b(nm) for nm in names])[:, None, None]

    S = s_ref[...]
    St = st_ref[...]
    yds = _conv3x3_scalar(jnp.stack([f1, f2]),
                          [wstack(("ds1", "ds2"), t) for t in range(9)])
    f1d = jnp.dot(S, jnp.dot(yds[0], St, preferred_element_type=jnp.float32),
                  preferred_element_type=jnp.float32) + emb_b("ds1")
    f2d = jnp.dot(S, jnp.dot(yds[1], St, preferred_element_type=jnp.float32),
                  preferred_element_type=jnp.float32) + emb_b("ds2")

    mb = ("m0", "b0", "m0", "b0")
    z = jnp.stack([f1d, f1d, f2d, f2d])
    z = _leaky(_conv3x3_scalar(z, [wstack(mb, t) for t in range(9)]) +
               bstack(mb), 0.01)
    mb = ("m1", "b1", "m1", "b1")
    z = _leaky(_conv3x3_scalar(z, [wstack(mb, t) for t in range(9)]) +
               bstack(mb), 0.01)
    mb = ("m2", "b2", "m2", "b2")
    z = _conv3x3_scalar(z, [wstack(mb, t) for t in range(9)]) + bstack(mb)
    is_m = (jax.lax.broadcasted_iota(jnp.int32, (4, 1, 1), 0) % 2) == 0
    z = jnp.where(is_m, jnp.tanh(z), z)
    mod2 = f2d * z[0] + z[1]
    mod1 = f1d * z[2] + z[3]
    fw2 = [jnp.stack([fusew_ref[row, t],
                      fusew_ref[row, 9 + t]])[:, None, None] for t in range(9)]
    yf = _conv3x3_scalar(jnp.stack([mod1, mod2]), fw2)
    fused = yf[0] + yf[1] + embb_ref[row, 8]

    U = u_ref[...]
    Ut = ut_ref[...]
    up = jnp.dot(U, jnp.dot(fused, Ut, preferred_element_type=jnp.float32),
                 preferred_element_type=jnp.float32)
    mask = 1.0 / (1.0 + jnp.exp(-up))
    e_in = jnp.maximum(h * mask[None, :, :], 0.0)

    eo = _dwconv3x3(e_in, coeff * c2w_ref[0], coeff * c2b_ref[0])
    first = flag_ref[0]

    @pl.when(first == 0)
    def _set():
        acc_ref[...] = eo

    @pl.when(first != 0)
    def _add():
        acc_ref[...] = acc_ref[...] + eo

    flag_ref[0] = 1


def _fusion_kernel(a_ref, w_ref, b_ref, o_ref):
    a = a_ref[0]
    o_ref[0] = jnp.dot(w_ref[...], a,
                       preferred_element_type=jnp.float32) + b_ref[...]


def _make_resize_mats():
    r = np.arange(_H)[:, None]
    c = np.arange(_H // 2)[None, :]
    k = r // 2
    even = (r % 2 == 0)
    U = (np.where(even & (c == k - 1), 0.25, 0.0) +
         np.where(even & (c == k), 0.75, 0.0) +
         np.where(~even & (c == k), 0.75, 0.0) +
         np.where(~even & (c == k + 1), 0.25, 0.0)).astype(np.float32)
    U[0, 0] += 0.25
    U[_H - 1, _H // 2 - 1] += 0.25
    rs = np.arange(_H // 2)[:, None]
    cs = np.arange(_H)[None, :]
    S = (cs == 2 * rs).astype(np.float32)
    return U, S


@jax.jit
def kernel(x, params):
    B = x.shape[0]
    groups = params["groups"]
    gates = params["gates"]

    c1w = jnp.stack([groups[g][e]["c1_w"].reshape(_DIM, 9)
                     for g in range(3) for e in range(4)])
    c1b = jnp.stack([groups[g][e]["c1_b"]
                     for g in range(3) for e in range(4)])[:, :, None]
    c2w = jnp.stack([groups[g][e]["c2_w"].reshape(_DIM, 9)
                     for g in range(3) for e in range(4)])
    c2b = jnp.stack([groups[g][e]["c2_b"]
                     for g in range(3) for e in range(4)])[:, :, None]
    embw = jnp.stack([
        jnp.concatenate([groups[g][e]["emb"][nm + "_w"].reshape(9)
                         for nm in _EMB_NAMES])
        for g in range(3) for e in range(4)])
    embb = jnp.stack([
        jnp.concatenate([jnp.stack([groups[g][e]["emb"][nm + "_b"][0]
                                    for nm in _EMB_NAMES]),
                         groups[g][e]["emb"]["fuse_b"]])
        for g in range(3) for e in range(4)])
    fusew = jnp.stack([groups[g][e]["emb"]["fuse_w"].reshape(18)
                       for g in range(3) for e in range(4)])
    fc0w = jnp.stack([gates[g]["fc0_w"] for g in range(3)])
    fc0b = jnp.stack([gates[g]["fc0_b"] for g in range(3)])
    fc1w = jnp.stack([gates[g]["fc1_w"] for g in range(3)])
    fc1b = jnp.stack([gates[g]["fc1_b"] for g in range(3)])

    U, S = _make_resize_mats()
    U = jnp.asarray(U)
    S = jnp.asarray(S)

    vm = pltpu.VMEM
    sm = pltpu.SMEM
    zero5 = lambda b, g, e: (0, 0, 0, 0, 0)

    inter = pl.pallas_call(
        _mofe_kernel,
        grid=(B, 3, 4),
        in_specs=[
            pl.BlockSpec((1, _DIM, _H, _H), lambda b, g, e: (b, 0, 0, 0)),
            pl.BlockSpec((1, _DIM, 9), lambda b, g, e: (g * 4 + e, 0, 0)),
            pl.BlockSpec((1, _DIM, 1), lambda b, g, e: (g * 4 + e, 0, 0)),
            pl.BlockSpec((1, _DIM, 9), lambda b, g, e: (g * 4 + e, 0, 0)),
            pl.BlockSpec((1, _DIM, 1), lambda b, g, e: (g * 4 + e, 0, 0)),
            pl.BlockSpec(memory_space=sm),
            pl.BlockSpec(memory_space=sm),
            pl.BlockSpec(memory_space=sm),
            pl.BlockSpec(memory_space=vm),
            pl.BlockSpec(memory_space=sm),
            pl.BlockSpec(memory_space=vm),
            pl.BlockSpec(memory_space=sm),
            pl.BlockSpec(memory_space=vm),
            pl.BlockSpec(memory_space=vm),
            pl.BlockSpec(memory_space=vm),
            pl.BlockSpec(memory_space=vm),
            pl.BlockSpec((1, _DIM, _DIM), lambda b, g, e: (g, 0, 0)),
            pl.BlockSpec(memory_space=vm),
        ],
        out_specs=pl.BlockSpec((1, _DIM, _H, _H),
                               lambda b, g, e: (b, 0, 0, 0)),
        out_shape=jax.ShapeDtypeStruct((B, _DIM, _H, _H), jnp.float32),
        scratch_shapes=[pltpu.SMEM((3, 4), jnp.float32),
                        pltpu.VMEM((_DIM, _H, _H), jnp.float32),
                        pltpu.SMEM((1,), jnp.int32)],
        compiler_params=pltpu.CompilerParams(
            dimension_semantics=("parallel", "arbitrary", "arbitrary"),
            vmem_limit_bytes=100 * 1024 * 1024),
    )(x, c1w, c1b, c2w, c2b, embw, embb, fusew,
      fc0w, fc0b, fc1w, fc1b, S, S.T, U, U.T,
      params["fusion_w"].reshape(_DIM, 3, _DIM).transpose(1, 0, 2),
      params["fusion_b"].reshape(_DIM, 1))

    return inter
